# baseline (speedup 1.0000x reference)
import functools

import jax
import jax.numpy as jnp
from jax import lax
from jax.experimental import pallas as pl
from jax.experimental.pallas import tpu as pltpu
from jax.experimental.pallas import tpu_sc as plsc

N_VARS = 50000
NPAD = 50048
E_EDGES = 1600000
NW = 32
SEG_PER_TILE = 1568
SP = SEG_PER_TILE * NW
CHUNK = 4096
EDGE_PAD = 2 * CHUNK + 16
NB = 48
SAMPLE = 128

_MESH = plsc.VectorSubcoreMesh(
    core_axis_name="c", subcore_axis_name="s", num_cores=2, num_subcores=16
)


def _seg_reduce_body(tab, ixin, ixout, bounds, out, tab_v, tab_sh, acc,
                     bi0, bo0, bi1, bo1, bnd_v, win_v, sem_t, sem0, sem1,
                     *, transform, apply_exp):
    sid = lax.axis_index("s")
    wid = sid * 2 + lax.axis_index("c")
    pltpu.sync_copy(bounds, bnd_v)
    seg_base = pl.multiple_of(wid * SEG_PER_TILE, 16)

    def refine(cb, bnd):
        w0 = pl.multiple_of(jnp.maximum(cb - 1, 0) * SAMPLE, 16)
        pltpu.sync_copy(ixout.at[pl.ds(w0, SAMPLE)], win_v)

        def cnt_body(i, c):
            w = win_v[pl.ds(i * 16, 16)]
            return c + jnp.sum(jnp.where(w < bnd, 1, 0))

        return w0 + lax.fori_loop(0, SAMPLE // 16, cnt_body, 0)

    e_lo = refine(bnd_v[pl.ds(wid, 16)][0], seg_base)
    e_hi = refine(bnd_v[pl.ds(wid + 1, 16)][0], seg_base + SEG_PER_TILE)

    zeros16 = jnp.zeros((16,), jnp.float32)

    def zero_body(i, _):
        acc[pl.ds(i * 16, 16)] = zeros16
        return 0

    lax.fori_loop(0, SEG_PER_TILE // 16, zero_body, 0)

    iot = lax.iota(jnp.int32, 16)
    is15 = iot == 15
    lt15 = iot < 15

    base = e_lo & ~15
    nch = (e_hi - base + CHUNK - 1) // CHUNK

    def chunk_off(k):
        pos = base + k * CHUNK
        return pl.multiple_of(jnp.minimum(pos, E_EDGES - CHUNK), 16)

    def start_dma(k, bi_, bo_, sem_):
        off = chunk_off(k)
        pltpu.async_copy(ixin.at[pl.ds(off, CHUNK)], bi_, sem_)
        pltpu.async_copy(ixout.at[pl.ds(off, CHUNK)],
                         bo_.at[pl.ds(0, CHUNK)], sem_)

    def wait_dma(bi_, bo_, sem_):
        pltpu.make_async_copy(ixin.at[pl.ds(0, CHUNK)], bi_, sem_).wait()
        pltpu.make_async_copy(ixout.at[pl.ds(0, CHUNK)],
                              bo_.at[pl.ds(0, CHUNK)], sem_).wait()

    @pl.when(nch > 0)
    def _():
        start_dma(0, bi0, bo0, sem0)

    @pl.when(sid == 0)
    def _():
        pltpu.sync_copy(tab, tab_v)
        pltpu.sync_copy(tab_v, tab_sh)

    plsc.subcore_barrier()

    @pl.when(sid != 0)
    def _():
        pltpu.sync_copy(tab_sh, tab_v)

    def compute_chunk(k, bi_, bo_):
        off = chunk_off(k)
        lo_k = jnp.maximum(e_lo, base + k * CHUNK)
        span = (e_hi - lo_k).astype(jnp.uint32)

        @plsc.parallel_loop(0, CHUNK, step=16, unroll=4)
        def vreg_body(j):
            io = bo_[pl.ds(j, 16)]
            nio = bo_[pl.ds(j + 1, 16)]
            ii = bi_[pl.ds(j, 16)]
            v = plsc.load_gather(tab_v, [transform(ii)])
            incl = plsc.cumsum(v)
            neq = io != nio
            rel = ((off + j - lo_k) + iot).astype(jnp.uint32)
            m_add = (neq | is15) & (rel < span)
            m_sub = neq & lt15 & ((rel + 1) < span)
            plsc.addupdate_scatter(acc, [io - seg_base], incl, mask=m_add)
            plsc.addupdate_scatter(acc, [nio - seg_base], -incl, mask=m_sub)

    def chunk_body(k, _):
        @pl.when(k % 2 == 0)
        def _():
            wait_dma(bi0, bo0, sem0)

            @pl.when(k + 1 < nch)
            def _():
                start_dma(k + 1, bi1, bo1, sem1)

            compute_chunk(k, bi0, bo0)

        @pl.when(k % 2 == 1)
        def _():
            wait_dma(bi1, bo1, sem1)

            @pl.when(k + 1 < nch)
            def _():
                start_dma(k + 1, bi0, bo0, sem0)

            compute_chunk(k, bi1, bo1)

        return 0

    lax.fori_loop(0, nch, chunk_body, 0)

    if apply_exp:
        def exp_body(i, _):
            acc[pl.ds(i * 16, 16)] = jnp.exp(acc[pl.ds(i * 16, 16)])
            return 0

        lax.fori_loop(0, SEG_PER_TILE // 16, exp_body, 0)

    pltpu.sync_copy(acc, out.at[pl.ds(seg_base, SEG_PER_TILE)])


def _transform_layer0(ii):
    j2 = ii - 2
    return (j2 >> 1) + jnp.where((j2 & 1) == 1, NPAD, 0)


def _make_seg_kernel(tab_len, transform, apply_exp):
    scratch = [
        pltpu.VMEM((tab_len,), jnp.float32),
        pltpu.VMEM_SHARED((tab_len,), jnp.float32),
        pltpu.VMEM((SEG_PER_TILE,), jnp.float32),
        pltpu.VMEM((CHUNK,), jnp.int32),
        pltpu.VMEM((CHUNK + 16,), jnp.int32),
        pltpu.VMEM((CHUNK,), jnp.int32),
        pltpu.VMEM((CHUNK + 16,), jnp.int32),
        pltpu.VMEM((NB,), jnp.int32),
        pltpu.VMEM((SAMPLE,), jnp.int32),
        pltpu.SemaphoreType.DMA,
        pltpu.SemaphoreType.DMA,
        pltpu.SemaphoreType.DMA,
    ]

    @functools.partial(
        pl.kernel,
        out_type=jax.ShapeDtypeStruct((SP,), jnp.float32),
        mesh=_MESH,
        scratch_types=scratch,
        compiler_params=pltpu.CompilerParams(needs_layout_passes=False),
    )
    def k(tab, ixin, ixout, bounds, out, tab_v, tab_sh, acc, bi0, bo0,
          bi1, bo1, bnd_v, win_v, sem_t, sem0, sem1):
        _seg_reduce_body(tab, ixin, ixout, bounds, out, tab_v, tab_sh, acc,
                         bi0, bo0, bi1, bo1, bnd_v, win_v, sem_t, sem0, sem1,
                         transform=transform, apply_exp=apply_exp)

    return k


_layer0 = _make_seg_kernel(2 * NPAD, _transform_layer0, True)
_layer1 = _make_seg_kernel(SP, lambda ii: ii, False)


def _log_table(x_pos):
    xp = jnp.pad(x_pos, (0, NPAD - N_VARS), constant_values=0.5)
    xp = xp.reshape(NPAD // 128, 128)

    def body(x_ref, o_ref):
        x = x_ref[...]
        o_ref[0] = jnp.log(x)
        o_ref[1] = jnp.log(1.0 - x)

    out = pl.pallas_call(
        body,
        out_shape=jax.ShapeDtypeStruct((2, NPAD // 128, 128), jnp.float32),
    )(xp)
    return out.reshape(-1)


def kernel(x_pos, ix_in0, ix_out0, ix_in1, ix_out1):
    ix_in0 = ix_in0.astype(jnp.int32)
    ix_out0 = ix_out0.astype(jnp.int32)
    ix_in1 = ix_in1.astype(jnp.int32)
    ix_out1 = ix_out1.astype(jnp.int32)

    ltab = _log_table(x_pos)

    seg_starts = jnp.arange(NW + 1, dtype=jnp.int32) * SEG_PER_TILE
    s0 = ix_out0[::SAMPLE]
    s1 = ix_out1[::SAMPLE]
    b0 = jnp.pad(
        jnp.sum(s0[:, None] < seg_starts[None, :], axis=0,
                dtype=jnp.int32), (0, NB - (NW + 1)))
    b1 = jnp.pad(
        jnp.sum(s1[:, None] < seg_starts[None, :], axis=0,
                dtype=jnp.int32), (0, NB - (NW + 1)))

    h0 = _layer0(ltab, ix_in0, ix_out0, b0)
    h1 = _layer1(h0, ix_in1, ix_out1, b1)
    return h1[:N_VARS]


# speedup vs baseline: 1.0089x; 1.0089x over previous
import functools

import jax
import jax.numpy as jnp
from jax import lax
from jax.experimental import pallas as pl
from jax.experimental.pallas import tpu as pltpu
from jax.experimental.pallas import tpu_sc as plsc

N_VARS = 50000
NPAD = 50048
E_EDGES = 1600000
NW = 32
SEG_PER_TILE = 1568
SP = SEG_PER_TILE * NW
CHUNK = 4096
EDGE_PAD = 2 * CHUNK + 16
G_GRAN = E_EDGES // 128
EMAIN = E_EDGES - 512
L1S = 196
L2S = 4

_GDN = lax.GatherDimensionNumbers(
    offset_dims=(), collapsed_slice_dims=(0,), start_index_map=(0,)
)

_MESH = plsc.VectorSubcoreMesh(
    core_axis_name="c", subcore_axis_name="s", num_cores=2, num_subcores=16
)


def _seg_reduce_body(tab, ixin, ixout2, out, tab_v, acc,
                     bi0, bo0, bi1, bo1, pidx_v, win2_v, win3_v,
                     sem_t, sem0, sem1, sem_g,
                     *, transform, apply_exp):
    wid = lax.axis_index("s") * 2 + lax.axis_index("c")
    tcopy = pltpu.async_copy(tab, tab_v, sem_t)
    seg_base = pl.multiple_of(wid * SEG_PER_TILE, 16)

    iot = lax.iota(jnp.int32, 16)
    zeros16i = jnp.zeros((16,), jnp.int32)

    def sample_heads(idx_fn):
        for c in range(4):
            pidx_v[pl.ds(c * 16, 16)] = idx_fn(iot + c * 16)
        pltpu.async_copy(ixout2.at[pidx_v], win2_v, sem_g).wait()

    def count_heads(bnd):
        t = 0
        for c in range(4):
            h = plsc.load_gather(win2_v, [iot + c * 16, zeros16i])
            t = t + jnp.sum(jnp.where(h < bnd, 1, 0))
        return t

    sample_heads(lambda k: jnp.minimum(k * L1S, G_GRAN - 1))
    c1_lo = count_heads(seg_base)
    c1_hi = count_heads(seg_base + SEG_PER_TILE)

    def refine(c1, bnd):
        g_lo = jnp.maximum(c1 - 1, 0) * L1S
        sample_heads(lambda k: jnp.minimum(g_lo + k * L2S, G_GRAN - 1))
        c2 = count_heads(bnd)
        g2u = jnp.minimum(g_lo + L2S * jnp.maximum(c2 - 1, 0), G_GRAN - 16)
        g2 = pl.multiple_of(g2u & ~7, 8)
        pltpu.sync_copy(ixout2.at[pl.ds(g2, 16), :], win3_v)

        def row_cnt(r, c):
            for s in range(8):
                w = win3_v[r, pl.ds(s * 16, 16)]
                c = c + jnp.sum(jnp.where(w < bnd, 1, 0))
            return c

        c = lax.fori_loop(0, 16, row_cnt, 0)
        pltpu.sync_copy(ixout2.at[pl.ds(G_GRAN - 4, 4), :],
                        win3_v.at[pl.ds(0, 4), :])
        t = lax.fori_loop(0, 4, row_cnt, 0)
        return g2 * 128 + c + jnp.where(c == 2048, t, 0)

    e_lo = refine(c1_lo, seg_base)
    e_hi = refine(c1_hi, seg_base + SEG_PER_TILE)

    zeros16 = jnp.zeros((16,), jnp.float32)

    def zero_body(i, _):
        acc[pl.ds(i * 16, 16)] = zeros16
        return 0

    lax.fori_loop(0, SEG_PER_TILE // 16, zero_body, 0)

    is15 = iot == 15
    lt15 = iot < 15
    nxt_l = jnp.minimum(iot + 1, 15)

    base = e_lo & ~1023
    e_hi_m = jnp.minimum(e_hi, EMAIN)
    nch = (e_hi_m - base + CHUNK - 1) // CHUNK

    def chunk_off(k):
        pos = base + k * CHUNK
        return pl.multiple_of(jnp.minimum(pos, EMAIN - CHUNK), 1024)

    def start_dma(k, bi_, bo_, sem_):
        off = chunk_off(k)
        pltpu.async_copy(ixin.at[pl.ds(off, CHUNK)], bi_, sem_)
        r0 = pl.multiple_of(off >> 7, 8)
        pltpu.async_copy(ixout2.at[pl.ds(r0, CHUNK // 128), :], bo_, sem_)

    def wait_dma(bi_, bo_, sem_):
        pltpu.make_async_copy(ixin.at[pl.ds(0, CHUNK)], bi_, sem_).wait()
        pltpu.make_async_copy(ixout2.at[pl.ds(0, CHUNK // 128), :],
                              bo_, sem_).wait()

    @pl.when(nch > 0)
    def _():
        start_dma(0, bi0, bo0, sem0)

    tcopy.wait()

    def vreg_math(io, ii, relbase, span):
        nio = lax.gather(io, nxt_l[:, None], _GDN, slice_sizes=(1,),
                         mode=lax.GatherScatterMode.PROMISE_IN_BOUNDS)
        v = plsc.load_gather(tab_v, [transform(ii)])
        incl = plsc.cumsum(v)
        neq = io != nio
        rel = (relbase + iot).astype(jnp.uint32)
        m_add = (neq | is15) & (rel < span)
        m_sub = neq & lt15 & ((rel + 1) < span)
        plsc.addupdate_scatter(acc, [io - seg_base], incl, mask=m_add)
        plsc.addupdate_scatter(acc, [nio - seg_base], -incl, mask=m_sub)

    def compute_chunk(k, bi_, bo_):
        off = chunk_off(k)
        lo_k = jnp.maximum(e_lo, base + k * CHUNK)
        span = (e_hi_m - lo_k).astype(jnp.uint32)

        @plsc.parallel_loop(0, CHUNK, step=16, unroll=4)
        def vreg_body(j):
            io = bo_[j >> 7, pl.ds(j & 127, 16)]
            ii = bi_[pl.ds(j, 16)]
            vreg_math(io, ii, (off + j - lo_k), span)

    def chunk_body(k, _):
        @pl.when(k % 2 == 0)
        def _():
            wait_dma(bi0, bo0, sem0)

            @pl.when(k + 1 < nch)
            def _():
                start_dma(k + 1, bi1, bo1, sem1)

            compute_chunk(k, bi0, bo0)

        @pl.when(k % 2 == 1)
        def _():
            wait_dma(bi1, bo1, sem1)

            @pl.when(k + 1 < nch)
            def _():
                start_dma(k + 1, bi0, bo0, sem0)

            compute_chunk(k, bi1, bo1)

        return 0

    lax.fori_loop(0, nch, chunk_body, 0)

    @pl.when(e_hi > EMAIN)
    def _():
        pltpu.sync_copy(ixin.at[pl.ds(EMAIN, 512)], bi0.at[pl.ds(0, 512)])
        pltpu.sync_copy(ixout2.at[pl.ds(G_GRAN - 4, 4), :],
                        win3_v.at[pl.ds(0, 4), :])
        lo_t = jnp.maximum(e_lo, EMAIN)
        span_t = (e_hi - lo_t).astype(jnp.uint32)
        for r in range(4):
            for s in range(8):
                j = r * 128 + s * 16
                io = win3_v[r, pl.ds(s * 16, 16)]
                ii = bi0[pl.ds(j, 16)]
                vreg_math(io, ii, (EMAIN + j - lo_t), span_t)

    if apply_exp:
        def exp_body(i, _):
            acc[pl.ds(i * 16, 16)] = jnp.exp(acc[pl.ds(i * 16, 16)])
            return 0

        lax.fori_loop(0, SEG_PER_TILE // 16, exp_body, 0)

    pltpu.sync_copy(acc, out.at[pl.ds(seg_base, SEG_PER_TILE)])


def _transform_layer0(ii):
    j2 = ii - 2
    return (j2 >> 1) + jnp.where((j2 & 1) == 1, NPAD, 0)


def _make_seg_kernel(tab_len, transform, apply_exp):
    scratch = [
        pltpu.VMEM((tab_len,), jnp.float32),
        pltpu.VMEM((SEG_PER_TILE,), jnp.float32),
        pltpu.VMEM((CHUNK,), jnp.int32),
        pltpu.VMEM((CHUNK // 128, 128), jnp.int32),
        pltpu.VMEM((CHUNK,), jnp.int32),
        pltpu.VMEM((CHUNK // 128, 128), jnp.int32),
        pltpu.VMEM((64,), jnp.int32),
        pltpu.VMEM((64, 128), jnp.int32),
        pltpu.VMEM((16, 128), jnp.int32),
        pltpu.SemaphoreType.DMA,
        pltpu.SemaphoreType.DMA,
        pltpu.SemaphoreType.DMA,
        pltpu.SemaphoreType.DMA,
    ]

    @functools.partial(
        pl.kernel,
        out_type=jax.ShapeDtypeStruct((SP,), jnp.float32),
        mesh=_MESH,
        scratch_types=scratch,
        compiler_params=pltpu.CompilerParams(needs_layout_passes=False),
    )
    def k(tab, ixin, ixout2, out, tab_v, acc, bi0, bo0,
          bi1, bo1, pidx_v, win2_v, win3_v, sem_t, sem0, sem1, sem_g):
        _seg_reduce_body(tab, ixin, ixout2, out, tab_v, acc,
                         bi0, bo0, bi1, bo1, pidx_v, win2_v, win3_v,
                         sem_t, sem0, sem1, sem_g,
                         transform=transform, apply_exp=apply_exp)

    return k


_layer0 = _make_seg_kernel(2 * NPAD, _transform_layer0, True)
_layer1 = _make_seg_kernel(SP, lambda ii: ii, False)


def _log_table(x_pos):
    xp = jnp.pad(x_pos, (0, NPAD - N_VARS), constant_values=0.5)
    xp = xp.reshape(NPAD // 128, 128)

    def body(x_ref, o_ref):
        x = x_ref[...]
        o_ref[0] = jnp.log(x)
        o_ref[1] = jnp.log(1.0 - x)

    out = pl.pallas_call(
        body,
        out_shape=jax.ShapeDtypeStruct((2, NPAD // 128, 128), jnp.float32),
    )(xp)
    return out.reshape(-1)


def kernel(x_pos, ix_in0, ix_out0, ix_in1, ix_out1):
    ix_in0 = ix_in0.astype(jnp.int32)
    ix_out0 = ix_out0.astype(jnp.int32)
    ix_in1 = ix_in1.astype(jnp.int32)
    ix_out1 = ix_out1.astype(jnp.int32)

    ltab = _log_table(x_pos)

    h0 = _layer0(ltab, ix_in0, ix_out0.reshape(G_GRAN, 128))
    h1 = _layer1(h0, ix_in1, ix_out1.reshape(G_GRAN, 128))
    return h1[:N_VARS]


# speedup vs baseline: 1.0613x; 1.0519x over previous
import functools

import jax
import jax.numpy as jnp
from jax import lax
from jax.experimental import pallas as pl
from jax.experimental.pallas import tpu as pltpu
from jax.experimental.pallas import tpu_sc as plsc

N_VARS = 50000
NPAD = 50048
E_EDGES = 1600000
NW = 32
SEG_PER_TILE = 1568
SP = SEG_PER_TILE * NW
CHUNK = 4096
EDGE_PAD = 2 * CHUNK + 16
G_GRAN = E_EDGES // 128
EMAIN = E_EDGES - 512
L1S = 196
L2S = 4

_GDN = lax.GatherDimensionNumbers(
    offset_dims=(), collapsed_slice_dims=(0,), start_index_map=(0,)
)

_MESH = plsc.VectorSubcoreMesh(
    core_axis_name="c", subcore_axis_name="s", num_cores=2, num_subcores=16
)


def _seg_reduce_body(tab, ixin, ixout2, out, tab_v, acc,
                     bi0, bo0, bi1, bo1, pidx_v, win2_v, win3_v,
                     sem_t, sem0, sem1, sem_g,
                     *, transform, apply_exp):
    wid = lax.axis_index("s") * 2 + lax.axis_index("c")
    tcopy = pltpu.async_copy(tab, tab_v, sem_t)
    seg_base = pl.multiple_of(wid * SEG_PER_TILE, 16)

    iot = lax.iota(jnp.int32, 16)
    zeros16i = jnp.zeros((16,), jnp.int32)

    def sample_heads(idx_fn):
        for c in range(4):
            pidx_v[pl.ds(c * 16, 16)] = idx_fn(iot + c * 16)
        pltpu.async_copy(ixout2.at[pidx_v], win2_v, sem_g).wait()

    def count_heads(bnd):
        t = 0
        for c in range(4):
            h = plsc.load_gather(win2_v, [iot + c * 16, zeros16i])
            t = t + jnp.sum(jnp.where(h < bnd, 1, 0))
        return t

    sample_heads(lambda k: jnp.minimum(k * L1S, G_GRAN - 1))
    c1_lo = count_heads(seg_base)
    c1_hi = count_heads(seg_base + SEG_PER_TILE)

    def refine(c1, bnd):
        g_lo = jnp.maximum(c1 - 1, 0) * L1S
        sample_heads(lambda k: jnp.minimum(g_lo + k * L2S, G_GRAN - 1))
        c2 = count_heads(bnd)
        g2 = jnp.minimum(g_lo + L2S * jnp.maximum(c2 - 1, 0), G_GRAN - 4)
        pidx_v[pl.ds(0, 16)] = jnp.minimum(g2 + iot, G_GRAN - 1)
        pltpu.async_copy(ixout2.at[pidx_v.at[pl.ds(0, 4)]], win3_v,
                         sem_g).wait()

        def row_cnt(r, c):
            for s in range(8):
                w = win3_v[r, pl.ds(s * 16, 16)]
                c = c + jnp.sum(jnp.where(w < bnd, 1, 0))
            return c

        return g2 * 128 + lax.fori_loop(0, 4, row_cnt, 0)

    e_lo = refine(c1_lo, seg_base)
    e_hi = refine(c1_hi, seg_base + SEG_PER_TILE)

    zeros16 = jnp.zeros((16,), jnp.float32)

    def zero_body(i, _):
        acc[pl.ds(i * 16, 16)] = zeros16
        return 0

    lax.fori_loop(0, SEG_PER_TILE // 16, zero_body, 0)

    is15 = iot == 15
    lt15 = iot < 15
    nxt_l = jnp.minimum(iot + 1, 15)

    base = e_lo & ~1023
    e_hi_m = jnp.minimum(e_hi, EMAIN)
    nch = (e_hi_m - base + CHUNK - 1) // CHUNK

    def chunk_off(k):
        pos = base + k * CHUNK
        return pl.multiple_of(jnp.minimum(pos, EMAIN - CHUNK), 1024)

    def start_dma(k, bi_, bo_, sem_):
        off = chunk_off(k)
        pltpu.async_copy(ixin.at[pl.ds(off, CHUNK)], bi_, sem_)
        r0 = pl.multiple_of(off >> 7, 8)
        pltpu.async_copy(ixout2.at[pl.ds(r0, CHUNK // 128), :], bo_, sem_)

    def wait_dma(bi_, bo_, sem_):
        pltpu.make_async_copy(ixin.at[pl.ds(0, CHUNK)], bi_, sem_).wait()
        pltpu.make_async_copy(ixout2.at[pl.ds(0, CHUNK // 128), :],
                              bo_, sem_).wait()

    @pl.when(nch > 0)
    def _():
        start_dma(0, bi0, bo0, sem0)

    tcopy.wait()

    def vreg_math(io, ii, relbase, span):
        nio = lax.gather(io, nxt_l[:, None], _GDN, slice_sizes=(1,),
                         mode=lax.GatherScatterMode.PROMISE_IN_BOUNDS)
        v = plsc.load_gather(tab_v, [transform(ii)])
        incl = plsc.cumsum(v)
        neq = io != nio
        rel = (relbase + iot).astype(jnp.uint32)
        m_add = (neq | is15) & (rel < span)
        m_sub = neq & lt15 & ((rel + 1) < span)
        plsc.addupdate_scatter(acc, [io - seg_base], incl, mask=m_add)
        plsc.addupdate_scatter(acc, [nio - seg_base], -incl, mask=m_sub)

    def compute_chunk(k, bi_, bo_):
        off = chunk_off(k)
        lo_k = jnp.maximum(e_lo, base + k * CHUNK)
        span = (e_hi_m - lo_k).astype(jnp.uint32)

        @plsc.parallel_loop(0, CHUNK, step=16, unroll=4)
        def vreg_body(j):
            io = bo_[j >> 7, pl.ds(j & 127, 16)]
            ii = bi_[pl.ds(j, 16)]
            vreg_math(io, ii, (off + j - lo_k), span)

    def chunk_body(k, _):
        @pl.when(k % 2 == 0)
        def _():
            wait_dma(bi0, bo0, sem0)

            @pl.when(k + 1 < nch)
            def _():
                start_dma(k + 1, bi1, bo1, sem1)

            compute_chunk(k, bi0, bo0)

        @pl.when(k % 2 == 1)
        def _():
            wait_dma(bi1, bo1, sem1)

            @pl.when(k + 1 < nch)
            def _():
                start_dma(k + 1, bi0, bo0, sem0)

            compute_chunk(k, bi1, bo1)

        return 0

    lax.fori_loop(0, nch, chunk_body, 0)

    @pl.when(e_hi > EMAIN)
    def _():
        pltpu.sync_copy(ixin.at[pl.ds(EMAIN, 512)], bi0.at[pl.ds(0, 512)])
        pltpu.sync_copy(ixout2.at[pl.ds(G_GRAN - 4, 4), :],
                        win3_v.at[pl.ds(0, 4), :])
        lo_t = jnp.maximum(e_lo, EMAIN)
        span_t = (e_hi - lo_t).astype(jnp.uint32)
        for r in range(4):
            for s in range(8):
                j = r * 128 + s * 16
                io = win3_v[r, pl.ds(s * 16, 16)]
                ii = bi0[pl.ds(j, 16)]
                vreg_math(io, ii, (EMAIN + j - lo_t), span_t)

    if apply_exp:
        def exp_body(i, _):
            acc[pl.ds(i * 16, 16)] = jnp.exp(acc[pl.ds(i * 16, 16)])
            return 0

        lax.fori_loop(0, SEG_PER_TILE // 16, exp_body, 0)

    pltpu.sync_copy(acc, out.at[pl.ds(seg_base, SEG_PER_TILE)])


def _transform_layer0(ii):
    j2 = ii - 2
    return (j2 >> 1) + jnp.where((j2 & 1) == 1, NPAD, 0)


def _make_seg_kernel(tab_len, transform, apply_exp):
    scratch = [
        pltpu.VMEM((tab_len,), jnp.float32),
        pltpu.VMEM((SEG_PER_TILE,), jnp.float32),
        pltpu.VMEM((CHUNK,), jnp.int32),
        pltpu.VMEM((CHUNK // 128, 128), jnp.int32),
        pltpu.VMEM((CHUNK,), jnp.int32),
        pltpu.VMEM((CHUNK // 128, 128), jnp.int32),
        pltpu.VMEM((64,), jnp.int32),
        pltpu.VMEM((64, 128), jnp.int32),
        pltpu.VMEM((4, 128), jnp.int32),
        pltpu.SemaphoreType.DMA,
        pltpu.SemaphoreType.DMA,
        pltpu.SemaphoreType.DMA,
        pltpu.SemaphoreType.DMA,
    ]

    @functools.partial(
        pl.kernel,
        out_type=jax.ShapeDtypeStruct((SP,), jnp.float32),
        mesh=_MESH,
        scratch_types=scratch,
        compiler_params=pltpu.CompilerParams(needs_layout_passes=False),
    )
    def k(tab, ixin, ixout2, out, tab_v, acc, bi0, bo0,
          bi1, bo1, pidx_v, win2_v, win3_v, sem_t, sem0, sem1, sem_g):
        _seg_reduce_body(tab, ixin, ixout2, out, tab_v, acc,
                         bi0, bo0, bi1, bo1, pidx_v, win2_v, win3_v,
                         sem_t, sem0, sem1, sem_g,
                         transform=transform, apply_exp=apply_exp)

    return k


_layer0 = _make_seg_kernel(2 * NPAD, _transform_layer0, True)
_layer1 = _make_seg_kernel(SP, lambda ii: ii, False)


def _log_table(x_pos):
    xp = jnp.pad(x_pos, (0, NPAD - N_VARS), constant_values=0.5)
    xp = xp.reshape(NPAD // 128, 128)

    def body(x_ref, o_ref):
        x = x_ref[...]
        o_ref[0] = jnp.log(x)
        o_ref[1] = jnp.log(1.0 - x)

    out = pl.pallas_call(
        body,
        out_shape=jax.ShapeDtypeStruct((2, NPAD // 128, 128), jnp.float32),
    )(xp)
    return out.reshape(-1)


def kernel(x_pos, ix_in0, ix_out0, ix_in1, ix_out1):
    ix_in0 = ix_in0.astype(jnp.int32)
    ix_out0 = ix_out0.astype(jnp.int32)
    ix_in1 = ix_in1.astype(jnp.int32)
    ix_out1 = ix_out1.astype(jnp.int32)

    ltab = _log_table(x_pos)

    h0 = _layer0(ltab, ix_in0, ix_out0.reshape(G_GRAN, 128))
    h1 = _layer1(h0, ix_in1, ix_out1.reshape(G_GRAN, 128))
    return h1[:N_VARS]
